# baseline (speedup 1.0000x reference)
import functools

import jax
import jax.numpy as jnp
import numpy as np
from jax import lax
from jax.experimental import pallas as pl
from jax.experimental.pallas import tpu as pltpu
from jax.experimental.pallas import tpu_sc as plsc

N = 10000
E = 320000
INP = 128
D = 256
NE = 4
NSTEPS = 8

NS = 16
NW = 2 * NS
ZROW = 312
ACC_T = 336
K = 128

_COL_PERM = np.empty((NE * D,), np.int32)
for _q in range(NE * D):
    _g, _r = divmod(_q % 32, 2)
    _COL_PERM[_q] = (_q // 32) * 32 + _g + (16 if _r else 0)

def _sc_mesh():
    return plsc.VectorSubcoreMesh(core_axis_name="c", subcore_axis_name="s",
                                  num_cores=2, num_subcores=NS)


def _sc_edge_phase(table, gidx, dst, meta):
    @functools.partial(
        pl.kernel,
        out_type=jax.ShapeDtypeStruct((N, D), jnp.float32),
        mesh=_sc_mesh(),
        scratch_types=[
            pltpu.VMEM((ACC_T, D), jnp.float32),
            pltpu.VMEM((K,), jnp.int32),
            pltpu.VMEM((2 * K,), jnp.int32),
            pltpu.VMEM((2, K, D // 2), jnp.int32),
            pltpu.VMEM((NW, 16), jnp.int32),
            pltpu.SemaphoreType.DMA,
        ],
        compiler_params=pltpu.CompilerParams(needs_layout_passes=False),
    )
    def k(table_h, gidx_h, dst_h, meta_h, out_h, acc, idxg, idxs2, rows2,
          metav, sem):
        c = lax.axis_index("c")
        s = lax.axis_index("s")
        t = c * NS + s

        def za(i, carry):
            r = i // 16
            col = (i % 16) * 16
            acc[r, pl.ds(col, 16)] = jnp.zeros((16,), jnp.float32)
            return carry

        lax.fori_loop(0, ACC_T * 16, za, 0)

        pltpu.sync_copy(meta_h, metav)
        mv = metav[t]
        lo = mv[0]
        hi = mv[1]
        base0 = (lo // K) * K
        nb = jnp.maximum(0, (hi - base0 + K - 1) // K)
        rbase = t * ZROW
        lo16 = jnp.full((16,), lo, jnp.int32)
        hi16 = jnp.full((16,), hi, jnp.int32)
        rb16 = jnp.full((16,), rbase, jnp.int32)
        dump16 = jnp.full((16,), ACC_T - 1, jnp.int32)
        lane = lax.iota(jnp.int32, 16)
        zvec = jnp.zeros((16,), jnp.float32)

        @pl.when(nb > 0)
        def _prologue():
            pltpu.sync_copy(gidx_h.at[pl.ds(base0, K)], idxg)
            pltpu.sync_copy(dst_h.at[pl.ds(base0, K)], idxs2.at[pl.ds(0, K)])
            pltpu.async_copy(table_h.at[idxg], rows2.at[0], sem)

        def body(i, carry):
            cur = carry[0]
            cv = list(carry[1:])
            b = i % 2
            base = base0 + i * K
            pltpu.make_async_copy(table_h.at[idxg], rows2.at[b], sem).wait()

            @pl.when(i + 1 < nb)
            def _prefetch():
                b2 = (i + 1) % 2
                pltpu.sync_copy(gidx_h.at[pl.ds(base + K, K)], idxg)
                pltpu.sync_copy(dst_h.at[pl.ds(base + K, K)],
                                idxs2.at[pl.ds(b2 * K, K)])
                pltpu.async_copy(table_h.at[idxg], rows2.at[b2], sem)

            def chunk(jc, ch):
                cur2 = ch[0]
                cv2 = list(ch[1:])
                v = idxs2[pl.ds(b * K + jc * 16, 16)]
                pos = jnp.full((16,), base, jnp.int32) + jc * 16 + lane
                ok = (pos >= lo16) & (pos < hi16)
                mvr = jnp.where(ok, v - rb16, dump16)
                for l in range(16):
                    r = mvr[l]
                    jrow = jc * 16 + l
                    prev = cur2

                    def _flush(pv=prev, cvs=tuple(cv2)):
                        for m in range(16):
                            sl = pl.ds(m * 16, 16)
                            acc[pv, sl] = acc[pv, sl] + cvs[m]
                        return (zvec,) * 16

                    def _keep(cvs=tuple(cv2)):
                        return cvs

                    cv2 = list(lax.cond(r != prev, _flush, _keep))
                    cur2 = r
                    for g in range(8):
                        w = rows2[b, jrow, pl.ds(g * 16, 16)]
                        even = plsc.bitcast(w << 16, jnp.float32)
                        odd = plsc.bitcast(
                            w & jnp.int32(-65536), jnp.float32)
                        cv2[2 * g] = cv2[2 * g] + even
                        cv2[2 * g + 1] = cv2[2 * g + 1] + odd
                return (cur2, *cv2)

            res = lax.fori_loop(0, K // 16, chunk, (cur, *cv))
            return res

        fin = lax.fori_loop(0, nb, body, (jnp.int32(ACC_T - 1),) + (zvec,) * 16)
        fcur = fin[0]
        for m in range(16):
            sl = pl.ds(m * 16, 16)
            acc[fcur, sl] = acc[fcur, sl] + fin[1 + m]

        @pl.when(t < NW - 1)
        def _write():
            for off, nrow in ((0, 128), (128, 128), (256, ZROW - 256)):
                pltpu.sync_copy(acc.at[pl.ds(off, nrow)],
                                out_h.at[pl.ds(rbase + off, nrow)])

        @pl.when(t == NW - 1)
        def _write_last():
            last = N - (NW - 1) * ZROW
            for off, nrow in ((0, 128), (128, 128), (256, last - 256)):
                pltpu.sync_copy(acc.at[pl.ds(off, nrow)],
                                out_h.at[pl.ds(rbase + off, nrow)])

    return k(table, gidx, dst, meta)


_GRID_R = 10
_BLK = N // _GRID_R


def _tablegen(h, Wcat, bcat):
    def body(h_ref, w_ref, b_ref, o_ref):
        o_ref[...] = (jnp.dot(h_ref[...], w_ref[...],
                              preferred_element_type=jnp.float32)
                      + b_ref[...]).astype(jnp.bfloat16)

    return pl.pallas_call(
        body,
        grid=(_GRID_R,),
        in_specs=[
            pl.BlockSpec((_BLK, D), lambda i: (i, 0)),
            pl.BlockSpec((D, NE * D), lambda i: (0, 0)),
            pl.BlockSpec((1, NE * D), lambda i: (0, 0)),
        ],
        out_specs=pl.BlockSpec((_BLK, NE * D), lambda i: (i, 0)),
        out_shape=jax.ShapeDtypeStruct((N, NE * D), jnp.bfloat16),
    )(h, Wcat, bcat)


def _gru_update(a, h, WihT, WhhT, bih, bhh, Wcat, bcat, want_table):
    def body(a_ref, h_ref, wih_ref, whh_ref, bih_ref, bhh_ref, wcat_ref,
             bcat_ref, hn_ref, p_ref):
        gi = jnp.dot(a_ref[...], wih_ref[...],
                     preferred_element_type=jnp.float32) + bih_ref[...]
        gh = jnp.dot(h_ref[...], whh_ref[...],
                     preferred_element_type=jnp.float32) + bhh_ref[...]
        r = 1.0 / (1.0 + jnp.exp(-(gi[:, :D] + gh[:, :D])))
        z = 1.0 / (1.0 + jnp.exp(-(gi[:, D:2 * D] + gh[:, D:2 * D])))
        n = jnp.tanh(gi[:, 2 * D:] + r * gh[:, 2 * D:])
        hn = (1.0 - z) * n + z * h_ref[...]
        hn_ref[...] = hn
        if p_ref is not None:
            p_ref[...] = (jnp.dot(hn, wcat_ref[...],
                                  preferred_element_type=jnp.float32)
                          + bcat_ref[...]).astype(jnp.bfloat16)

    if want_table:
        out_shape = (jax.ShapeDtypeStruct((N, D), jnp.float32),
                     jax.ShapeDtypeStruct((N, NE * D), jnp.bfloat16))
        out_specs = (pl.BlockSpec((_BLK, D), lambda i: (i, 0)),
                     pl.BlockSpec((_BLK, NE * D), lambda i: (i, 0)))
        fn = body
    else:
        out_shape = jax.ShapeDtypeStruct((N, D), jnp.float32)
        out_specs = pl.BlockSpec((_BLK, D), lambda i: (i, 0))

        def fn(a_ref, h_ref, wih_ref, whh_ref, bih_ref, bhh_ref, wcat_ref,
               bcat_ref, hn_ref):
            body(a_ref, h_ref, wih_ref, whh_ref, bih_ref, bhh_ref, wcat_ref,
                 bcat_ref, hn_ref, None)

    return pl.pallas_call(
        fn,
        grid=(_GRID_R,),
        in_specs=[
            pl.BlockSpec((_BLK, D), lambda i: (i, 0)),
            pl.BlockSpec((_BLK, D), lambda i: (i, 0)),
            pl.BlockSpec((D, 3 * D), lambda i: (0, 0)),
            pl.BlockSpec((D, 3 * D), lambda i: (0, 0)),
            pl.BlockSpec((1, 3 * D), lambda i: (0, 0)),
            pl.BlockSpec((1, 3 * D), lambda i: (0, 0)),
            pl.BlockSpec((D, NE * D), lambda i: (0, 0)),
            pl.BlockSpec((1, NE * D), lambda i: (0, 0)),
        ],
        out_specs=out_specs,
        out_shape=out_shape,
    )(a, h, WihT, WhhT, bih, bhh, Wcat, bcat)


def _pool_cls(h, x, Wc, bc):
    def body(h_ref, x_ref, wc_ref, bc_ref, o_ref, acc_ref):
        i = pl.program_id(0)

        @pl.when(i == 0)
        def _():
            acc_ref[...] = jnp.zeros_like(acc_ref)

        acc_ref[0:1, :D] += jnp.sum(h_ref[...], axis=0, keepdims=True)
        acc_ref[0:1, D:] += jnp.sum(x_ref[...], axis=0, keepdims=True)

        @pl.when(i == _GRID_R - 1)
        def _():
            logit = jnp.sum(acc_ref[0:1, :] * wc_ref[...], axis=1,
                            keepdims=True) + bc_ref[...]
            o_ref[...] = 1.0 / (1.0 + jnp.exp(-logit))

    return pl.pallas_call(
        body,
        grid=(_GRID_R,),
        in_specs=[
            pl.BlockSpec((_BLK, D), lambda i: (i, 0)),
            pl.BlockSpec((_BLK, INP), lambda i: (i, 0)),
            pl.BlockSpec((1, D + INP), lambda i: (0, 0)),
            pl.BlockSpec((1, 1), lambda i: (0, 0)),
        ],
        out_specs=pl.BlockSpec((1, 1), lambda i: (0, 0)),
        out_shape=jax.ShapeDtypeStruct((1, 1), jnp.float32),
        scratch_shapes=[pltpu.VMEM((8, D + INP), jnp.float32)],
    )(h, x, Wc, bc)


def _prep_edges(edge_index, edge_type):
    src = edge_index[0]
    dst = edge_index[1]
    gidx = src * NE + edge_type
    dst_s, gidx_s = lax.sort((dst, gidx), num_keys=1)
    targets = jnp.concatenate(
        [jnp.arange(NW, dtype=jnp.int32) * ZROW,
         jnp.array([N], jnp.int32)])
    bnd = jnp.searchsorted(dst_s, targets, side="left").astype(jnp.int32)
    meta = jnp.zeros((NW, 16), jnp.int32)
    meta = meta.at[:, 0].set(bnd[:NW]).at[:, 1].set(bnd[1:])
    return gidx_s, dst_s, meta


def kernel(x, edge_index, edge_type, W_et, b_et, W_ih, W_hh, b_ih, b_hh,
           W_cls, b_cls):
    Wcat = jnp.transpose(W_et, (2, 0, 1)).reshape(D, NE * D)[:, _COL_PERM]
    bcat = b_et.reshape(1, NE * D)[:, _COL_PERM]
    WihT = W_ih.T
    WhhT = W_hh.T
    bih = b_ih.reshape(1, 3 * D)
    bhh = b_hh.reshape(1, 3 * D)
    Wc = W_cls.reshape(1, D + INP)
    bc = b_cls.reshape(1, 1)

    egs, eds, meta = _prep_edges(edge_index, edge_type)

    h = jnp.concatenate([x, jnp.zeros((N, D - INP), x.dtype)], axis=1)
    P = _tablegen(h, Wcat, bcat)
    for step in range(NSTEPS):
        tab_i32 = lax.bitcast_convert_type(
            P.reshape(N * NE, D // 2, 2), jnp.int32)
        a = _sc_edge_phase(tab_i32, egs, eds, meta)
        res = _gru_update(a, h, WihT, WhhT, bih, bhh, Wcat, bcat,
                          want_table=(step < NSTEPS - 1))
        if step < NSTEPS - 1:
            h, P = res
        else:
            h = res
    return _pool_cls(h, x, Wc, bc)


# speedup vs baseline: 10.0902x; 10.0902x over previous
import functools

import jax
import jax.numpy as jnp
import numpy as np
from jax import lax
from jax.experimental import pallas as pl
from jax.experimental.pallas import tpu as pltpu
from jax.experimental.pallas import tpu_sc as plsc

N = 10000
E = 320000
INP = 128
D = 256
NE = 4
NSTEPS = 8

NS = 16
NW = 2 * NS
ZROW = 312
ACC_T = 336
K = 128

_COL_PERM = np.empty((NE * D,), np.int32)
for _q in range(NE * D):
    _g, _r = divmod(_q % 32, 2)
    _COL_PERM[_q] = (_q // 32) * 32 + _g + (16 if _r else 0)

def _sc_mesh():
    return plsc.VectorSubcoreMesh(core_axis_name="c", subcore_axis_name="s",
                                  num_cores=2, num_subcores=NS)


def _sc_edge_phase(table, gidx, dst, meta):
    @functools.partial(
        pl.kernel,
        out_type=jax.ShapeDtypeStruct((N, D), jnp.float32),
        mesh=_sc_mesh(),
        scratch_types=[
            pltpu.VMEM((ACC_T, D), jnp.float32),
            pltpu.VMEM((K,), jnp.int32),
            pltpu.VMEM((2 * K,), jnp.int32),
            pltpu.VMEM((2, K, D // 2), jnp.int32),
            pltpu.VMEM((NW, 16), jnp.int32),
            pltpu.SemaphoreType.DMA,
        ],
        compiler_params=pltpu.CompilerParams(needs_layout_passes=False),
    )
    def k(table_h, gidx_h, dst_h, meta_h, out_h, acc, idxg, idxs2, rows2,
          metav, sem):
        c = lax.axis_index("c")
        s = lax.axis_index("s")
        t = c * NS + s

        def za(i, carry):
            r = i // 16
            col = (i % 16) * 16
            acc[r, pl.ds(col, 16)] = jnp.zeros((16,), jnp.float32)
            return carry

        lax.fori_loop(0, ACC_T * 16, za, 0)

        pltpu.sync_copy(meta_h, metav)
        mv = metav[t]
        lo = mv[0]
        hi = mv[1]
        base0 = (lo // K) * K
        nb = jnp.maximum(0, (hi - base0 + K - 1) // K)
        rbase = t * ZROW
        lo16 = jnp.full((16,), lo, jnp.int32)
        hi16 = jnp.full((16,), hi, jnp.int32)
        rb16 = jnp.full((16,), rbase, jnp.int32)
        dump16 = jnp.full((16,), ACC_T - 1, jnp.int32)
        lane = lax.iota(jnp.int32, 16)
        zvec = jnp.zeros((16,), jnp.float32)

        @pl.when(nb > 0)
        def _prologue():
            pltpu.sync_copy(gidx_h.at[pl.ds(base0, K)], idxg)
            pltpu.sync_copy(dst_h.at[pl.ds(base0, K)], idxs2.at[pl.ds(0, K)])
            pltpu.async_copy(table_h.at[idxg], rows2.at[0], sem)

        def body(i, carry):
            cur = carry[0]
            cv = list(carry[1:])
            b = i % 2
            base = base0 + i * K
            pltpu.make_async_copy(table_h.at[idxg], rows2.at[b], sem).wait()

            @pl.when(i + 1 < nb)
            def _prefetch():
                b2 = (i + 1) % 2
                pltpu.sync_copy(gidx_h.at[pl.ds(base + K, K)], idxg)
                pltpu.sync_copy(dst_h.at[pl.ds(base + K, K)],
                                idxs2.at[pl.ds(b2 * K, K)])
                pltpu.async_copy(table_h.at[idxg], rows2.at[b2], sem)

            def chunk(jc, ch):
                cur2 = ch[0]
                cv2 = list(ch[1:])
                v = idxs2[pl.ds(b * K + jc * 16, 16)]
                pos = jnp.full((16,), base, jnp.int32) + jc * 16 + lane
                ok = (pos >= lo16) & (pos < hi16)
                mvr = jnp.where(ok, v - rb16, dump16)
                for l in range(16):
                    r = mvr[l]
                    jrow = jc * 16 + l
                    prev = cur2

                    def _flush(pv=prev, cvs=tuple(cv2)):
                        for m in range(16):
                            sl = pl.ds(m * 16, 16)
                            acc[pv, sl] = acc[pv, sl] + cvs[m]
                        return (zvec,) * 16

                    def _keep(cvs=tuple(cv2)):
                        return cvs

                    cv2 = list(lax.cond(r != prev, _flush, _keep))
                    cur2 = r
                    for g in range(8):
                        w = rows2[b, jrow, pl.ds(g * 16, 16)]
                        even = plsc.bitcast(w << 16, jnp.float32)
                        odd = plsc.bitcast(
                            w & jnp.int32(-65536), jnp.float32)
                        cv2[2 * g] = cv2[2 * g] + even
                        cv2[2 * g + 1] = cv2[2 * g + 1] + odd
                return (cur2, *cv2)

            res = lax.fori_loop(0, K // 16, chunk, (cur, *cv))
            return res

        fin = lax.fori_loop(0, nb, body, (jnp.int32(ACC_T - 1),) + (zvec,) * 16)
        fcur = fin[0]
        for m in range(16):
            sl = pl.ds(m * 16, 16)
            acc[fcur, sl] = acc[fcur, sl] + fin[1 + m]

        @pl.when(t < NW - 1)
        def _write():
            for off, nrow in ((0, 128), (128, 128), (256, ZROW - 256)):
                pltpu.sync_copy(acc.at[pl.ds(off, nrow)],
                                out_h.at[pl.ds(rbase + off, nrow)])

        @pl.when(t == NW - 1)
        def _write_last():
            last = N - (NW - 1) * ZROW
            for off, nrow in ((0, 128), (128, 128), (256, last - 256)):
                pltpu.sync_copy(acc.at[pl.ds(off, nrow)],
                                out_h.at[pl.ds(rbase + off, nrow)])

    return k(table, gidx, dst, meta)


_GRID_R = 10
_BLK = N // _GRID_R


def _pack_bf16_pair(xe, xo):
    ue = lax.bitcast_convert_type(xe, jnp.uint32)
    uo = lax.bitcast_convert_type(xo, jnp.uint32)
    re = (ue + jnp.uint32(0x7FFF) + ((ue >> 16) & jnp.uint32(1)))
    ro = (uo + jnp.uint32(0x7FFF) + ((uo >> 16) & jnp.uint32(1)))
    word = ((re & jnp.uint32(0xFFFF0000)) >> 16) | (ro & jnp.uint32(0xFFFF0000))
    return lax.bitcast_convert_type(word, jnp.int32)


def _tablegen(h, We, Wo, be, bo):
    def body(h_ref, we_ref, wo_ref, be_ref, bo_ref, o_ref):
        xe = jnp.dot(h_ref[...], we_ref[...],
                     preferred_element_type=jnp.float32) + be_ref[...]
        xo = jnp.dot(h_ref[...], wo_ref[...],
                     preferred_element_type=jnp.float32) + bo_ref[...]
        o_ref[...] = _pack_bf16_pair(xe, xo)

    return pl.pallas_call(
        body,
        grid=(_GRID_R,),
        in_specs=[
            pl.BlockSpec((_BLK, D), lambda i: (i, 0)),
            pl.BlockSpec((D, NE * D // 2), lambda i: (0, 0)),
            pl.BlockSpec((D, NE * D // 2), lambda i: (0, 0)),
            pl.BlockSpec((1, NE * D // 2), lambda i: (0, 0)),
            pl.BlockSpec((1, NE * D // 2), lambda i: (0, 0)),
        ],
        out_specs=pl.BlockSpec((_BLK, NE * D // 2), lambda i: (i, 0)),
        out_shape=jax.ShapeDtypeStruct((N, NE * D // 2), jnp.int32),
    )(h, We, Wo, be, bo)


def _gru_update(a, h, WihT, WhhT, bih, bhh, We, Wo, be, bo, want_table):
    def body(a_ref, h_ref, wih_ref, whh_ref, bih_ref, bhh_ref, we_ref,
             wo_ref, be_ref, bo_ref, hn_ref, p_ref):
        gi = jnp.dot(a_ref[...], wih_ref[...],
                     preferred_element_type=jnp.float32) + bih_ref[...]
        gh = jnp.dot(h_ref[...], whh_ref[...],
                     preferred_element_type=jnp.float32) + bhh_ref[...]
        r = 1.0 / (1.0 + jnp.exp(-(gi[:, :D] + gh[:, :D])))
        z = 1.0 / (1.0 + jnp.exp(-(gi[:, D:2 * D] + gh[:, D:2 * D])))
        n = jnp.tanh(gi[:, 2 * D:] + r * gh[:, 2 * D:])
        hn = (1.0 - z) * n + z * h_ref[...]
        hn_ref[...] = hn
        if p_ref is not None:
            xe = jnp.dot(hn, we_ref[...],
                         preferred_element_type=jnp.float32) + be_ref[...]
            xo = jnp.dot(hn, wo_ref[...],
                         preferred_element_type=jnp.float32) + bo_ref[...]
            p_ref[...] = _pack_bf16_pair(xe, xo)

    if want_table:
        out_shape = (jax.ShapeDtypeStruct((N, D), jnp.float32),
                     jax.ShapeDtypeStruct((N, NE * D // 2), jnp.int32))
        out_specs = (pl.BlockSpec((_BLK, D), lambda i: (i, 0)),
                     pl.BlockSpec((_BLK, NE * D // 2), lambda i: (i, 0)))
        fn = body
    else:
        out_shape = jax.ShapeDtypeStruct((N, D), jnp.float32)
        out_specs = pl.BlockSpec((_BLK, D), lambda i: (i, 0))

        def fn(a_ref, h_ref, wih_ref, whh_ref, bih_ref, bhh_ref, we_ref,
               wo_ref, be_ref, bo_ref, hn_ref):
            body(a_ref, h_ref, wih_ref, whh_ref, bih_ref, bhh_ref, we_ref,
                 wo_ref, be_ref, bo_ref, hn_ref, None)

    return pl.pallas_call(
        fn,
        grid=(_GRID_R,),
        in_specs=[
            pl.BlockSpec((_BLK, D), lambda i: (i, 0)),
            pl.BlockSpec((_BLK, D), lambda i: (i, 0)),
            pl.BlockSpec((D, 3 * D), lambda i: (0, 0)),
            pl.BlockSpec((D, 3 * D), lambda i: (0, 0)),
            pl.BlockSpec((1, 3 * D), lambda i: (0, 0)),
            pl.BlockSpec((1, 3 * D), lambda i: (0, 0)),
            pl.BlockSpec((D, NE * D // 2), lambda i: (0, 0)),
            pl.BlockSpec((D, NE * D // 2), lambda i: (0, 0)),
            pl.BlockSpec((1, NE * D // 2), lambda i: (0, 0)),
            pl.BlockSpec((1, NE * D // 2), lambda i: (0, 0)),
        ],
        out_specs=out_specs,
        out_shape=out_shape,
    )(a, h, WihT, WhhT, bih, bhh, We, Wo, be, bo)


def _pool_cls(h, x, Wc, bc):
    def body(h_ref, x_ref, wc_ref, bc_ref, o_ref, acc_ref):
        i = pl.program_id(0)

        @pl.when(i == 0)
        def _():
            acc_ref[...] = jnp.zeros_like(acc_ref)

        acc_ref[0:1, :D] += jnp.sum(h_ref[...], axis=0, keepdims=True)
        acc_ref[0:1, D:] += jnp.sum(x_ref[...], axis=0, keepdims=True)

        @pl.when(i == _GRID_R - 1)
        def _():
            logit = jnp.sum(acc_ref[0:1, :] * wc_ref[...], axis=1,
                            keepdims=True) + bc_ref[...]
            o_ref[...] = 1.0 / (1.0 + jnp.exp(-logit))

    return pl.pallas_call(
        body,
        grid=(_GRID_R,),
        in_specs=[
            pl.BlockSpec((_BLK, D), lambda i: (i, 0)),
            pl.BlockSpec((_BLK, INP), lambda i: (i, 0)),
            pl.BlockSpec((1, D + INP), lambda i: (0, 0)),
            pl.BlockSpec((1, 1), lambda i: (0, 0)),
        ],
        out_specs=pl.BlockSpec((1, 1), lambda i: (0, 0)),
        out_shape=jax.ShapeDtypeStruct((1, 1), jnp.float32),
        scratch_shapes=[pltpu.VMEM((8, D + INP), jnp.float32)],
    )(h, x, Wc, bc)


def _prep_edges(edge_index, edge_type):
    src = edge_index[0]
    dst = edge_index[1]
    gidx = src * NE + edge_type
    dst_s, gidx_s = lax.sort((dst, gidx), num_keys=1)
    targets = jnp.concatenate(
        [jnp.arange(NW, dtype=jnp.int32) * ZROW,
         jnp.array([N], jnp.int32)])
    bnd = jnp.searchsorted(dst_s, targets, side="left").astype(jnp.int32)
    meta = jnp.zeros((NW, 16), jnp.int32)
    meta = meta.at[:, 0].set(bnd[:NW]).at[:, 1].set(bnd[1:])
    return gidx_s, dst_s, meta


def kernel(x, edge_index, edge_type, W_et, b_et, W_ih, W_hh, b_ih, b_hh,
           W_cls, b_cls):
    Wcat = jnp.transpose(W_et, (2, 0, 1)).reshape(D, NE * D)
    bcat = b_et.reshape(1, NE * D)
    We = Wcat[:, _COL_PERM[0::2]]
    Wo = Wcat[:, _COL_PERM[1::2]]
    be = bcat[:, _COL_PERM[0::2]]
    bo = bcat[:, _COL_PERM[1::2]]
    WihT = W_ih.T
    WhhT = W_hh.T
    bih = b_ih.reshape(1, 3 * D)
    bhh = b_hh.reshape(1, 3 * D)
    Wc = W_cls.reshape(1, D + INP)
    bc = b_cls.reshape(1, 1)

    egs, eds, meta = _prep_edges(edge_index, edge_type)

    h = jnp.concatenate([x, jnp.zeros((N, D - INP), x.dtype)], axis=1)
    P = _tablegen(h, We, Wo, be, bo)
    for step in range(NSTEPS):
        a = _sc_edge_phase(P.reshape(N * NE, D // 2), egs, eds, meta)
        res = _gru_update(a, h, WihT, WhhT, bih, bhh, We, Wo, be, bo,
                          want_table=(step < NSTEPS - 1))
        if step < NSTEPS - 1:
            h, P = res
        else:
            h = res
    return _pool_cls(h, x, Wc, bc)
